# baseline (device time: 98286 ns/iter reference)
import jax
import jax.numpy as jnp
from jax import lax
from jax.experimental import pallas as pl
from jax.experimental.pallas import tpu as pltpu

N_DEV = 4
N_SUB = 4


def kernel(O, Wo):
    b, s, h, d = O.shape
    k = h * d
    n = Wo.shape[1]
    nh = n // 2
    s_chunk = s // N_DEV
    o = O.reshape(b, s, k)
    assert b == N_SUB

    def body(o_ref, wo_ref, out_ref, ob_ref, wob_ref, cw_ref, ccw_ref,
             cw_ssems, cw_rsems, ccw_ssems, ccw_rsems):
        my = lax.axis_index("i")
        left = lax.rem(my + N_DEV - 1, N_DEV)
        right = lax.rem(my + 1, N_DEV)

        for bb in range(b):
            ob_ref[bb] = o_ref[bb].astype(jnp.bfloat16)
        wob_ref[...] = wo_ref[...].astype(jnp.bfloat16)

        barrier_sem = pltpu.get_barrier_semaphore()
        for nbr in (left, right):
            pl.semaphore_signal(
                barrier_sem, inc=1,
                device_id=(nbr,), device_id_type=pl.DeviceIdType.MESH,
            )
        pl.semaphore_wait(barrier_sem, 2)

        def pdot(c, col0, bb):
            return jnp.dot(
                ob_ref[bb, pl.ds(c * s_chunk, s_chunk), :],
                wob_ref[:, col0:col0 + nh],
                preferred_element_type=jnp.float32,
            )

        def mk(dir_ref, ssems, rsems, t, j, dev):
            sl = pl.ds(j, 1)
            return pltpu.make_async_remote_copy(
                src_ref=dir_ref.at[t, sl],
                dst_ref=dir_ref.at[t + 1, sl],
                send_sem=ssems.at[t, j],
                recv_sem=rsems.at[t, j],
                device_id=(dev,),
                device_id_type=pl.DeviceIdType.MESH,
            )

        for j in range(N_SUB):
            cw_ref[0, j] = pdot(left, 0, j).astype(jnp.bfloat16)
            mk(cw_ref, cw_ssems, cw_rsems, 0, j, right).start()
            ccw_ref[0, j] = pdot(right, nh, j).astype(jnp.bfloat16)
            mk(ccw_ref, ccw_ssems, ccw_rsems, 0, j, left).start()

        for t in range(N_DEV - 1):
            c_cw = lax.rem(my + (N_DEV + 2 - t), N_DEV)
            c_ccw = lax.rem(my + 2 + t, N_DEV)
            for j in range(N_SUB):
                mk(cw_ref, cw_ssems, cw_rsems, t, j, right).wait_recv()
                if t < N_DEV - 2:
                    cw_ref[t + 1, j] = (
                        pdot(c_cw, 0, j) + cw_ref[t + 1, j]
                    ).astype(jnp.bfloat16)
                    mk(cw_ref, cw_ssems, cw_rsems, t + 1, j, right).start()
                else:
                    out_ref[j, :, 0:nh] = pdot(c_cw, 0, j) + cw_ref[t + 1, j]
                mk(ccw_ref, ccw_ssems, ccw_rsems, t, j, left).wait_recv()
                if t < N_DEV - 2:
                    ccw_ref[t + 1, j] = (
                        pdot(c_ccw, nh, j) + ccw_ref[t + 1, j]
                    ).astype(jnp.bfloat16)
                    mk(ccw_ref, ccw_ssems, ccw_rsems, t + 1, j, left).start()
                else:
                    out_ref[j, :, nh:n] = pdot(c_ccw, nh, j) + ccw_ref[t + 1, j]

        for t in range(N_DEV - 1):
            for j in range(N_SUB):
                mk(cw_ref, cw_ssems, cw_rsems, t, j, right).wait_send()
                mk(ccw_ref, ccw_ssems, ccw_rsems, t, j, left).wait_send()

    out_shape = jax.ShapeDtypeStruct((b, s_chunk, n), jnp.float32)
    return pl.pallas_call(
        body,
        out_shape=out_shape,
        in_specs=[
            pl.BlockSpec(memory_space=pltpu.VMEM),
            pl.BlockSpec(memory_space=pltpu.VMEM),
        ],
        out_specs=pl.BlockSpec(memory_space=pltpu.VMEM),
        scratch_shapes=[
            pltpu.VMEM((b, s, k), jnp.bfloat16),
            pltpu.VMEM((k, n), jnp.bfloat16),
            pltpu.VMEM((N_DEV, b, s_chunk, nh), jnp.bfloat16),
            pltpu.VMEM((N_DEV, b, s_chunk, nh), jnp.bfloat16),
            pltpu.SemaphoreType.DMA((N_DEV - 1, N_SUB)),
            pltpu.SemaphoreType.DMA((N_DEV - 1, N_SUB)),
            pltpu.SemaphoreType.DMA((N_DEV - 1, N_SUB)),
            pltpu.SemaphoreType.DMA((N_DEV - 1, N_SUB)),
        ],
        compiler_params=pltpu.CompilerParams(
            collective_id=0,
            vmem_limit_bytes=100 * 1024 * 1024,
        ),
    )(o, Wo)
